# baseline (speedup 1.0000x reference)
import math

import jax
import jax.numpy as jnp
from jax.experimental import pallas as pl
from jax.experimental.pallas import tpu as pltpu

N = 100000
K = 16
E = N * K
TAU = 2.0
GAMMA = -0.1
ZETA = 1.1
EPS = 1e-06
_C = math.log((0.0 - GAMMA) / (ZETA - 0.0) + EPS)

BCE = 160000
GJ = E // BCE
BWT = 524288


def _gen_kernel(logit_ref, edges_ref, weights_ref, pen_ref, s0_ref):
    j = pl.program_id(0)
    logit = logit_ref[0]

    @pl.when(j == 0)
    def _():
        c = jax.lax.broadcasted_iota(jnp.int32, (1, BCE), 1)
        srcrow = c >> 4
        s0_ref[0:1, :] = srcrow
        s0_ref[1:2, :] = srcrow + ((c & (K - 1)) + 1)
        pen_ref[0] = jax.nn.sigmoid(logit - TAU * _C)

    @pl.when((j >= 1) & (j < 5))
    def _():
        s = jax.nn.sigmoid(logit / TAU)
        gate = jnp.clip(s * (ZETA - GAMMA) + GAMMA, 0.0, 1.0)
        weights_ref[...] = jnp.full((BWT,), gate, dtype=jnp.float32)

    v = s0_ref[...] + j * (BCE // K)

    @pl.when(j < GJ - 1)
    def _():
        edges_ref[...] = v

    @pl.when(j == GJ - 1)
    def _():
        edges_ref[...] = jnp.where(v >= N, v - N, v)


def kernel(x, batch, logit):
    del x, batch
    edges, weights, pen = pl.pallas_call(
        _gen_kernel,
        grid=(GJ,),
        in_specs=[pl.BlockSpec(memory_space=pltpu.SMEM)],
        out_specs=[
            pl.BlockSpec((2, BCE), lambda j: (0, j)),
            pl.BlockSpec((BWT,), lambda j: (jnp.clip(j - 1, 0, 3),)),
            pl.BlockSpec(memory_space=pltpu.SMEM),
        ],
        out_shape=[
            jax.ShapeDtypeStruct((2, E), jnp.int32),
            jax.ShapeDtypeStruct((E,), jnp.float32),
            jax.ShapeDtypeStruct((1,), jnp.float32),
        ],
        scratch_shapes=[pltpu.VMEM((2, BCE), jnp.int32)],
    )(logit)
    return edges, weights, pen.reshape(())


# speedup vs baseline: 1.0155x; 1.0155x over previous
import math

import jax
import jax.numpy as jnp
from jax.experimental import pallas as pl
from jax.experimental.pallas import tpu as pltpu

N = 100000
K = 16
E = N * K
TAU = 2.0
GAMMA = -0.1
ZETA = 1.1
EPS = 1e-06
_C = math.log((0.0 - GAMMA) / (ZETA - 0.0) + EPS)

BCE = 160000
GJ = E // BCE
BWT = 524288


def _gen_kernel(logit_ref, edges_ref, weights_ref, pen_ref, s0_ref):
    j = pl.program_id(0)
    logit = logit_ref[0]

    @pl.when(j == 0)
    def _():
        c = jax.lax.broadcasted_iota(jnp.int32, (2, BCE), 1)
        row = jax.lax.broadcasted_iota(jnp.int32, (2, BCE), 0)
        s0_ref[...] = (c >> 4) + jnp.where(row == 0, 0, (c & (K - 1)) + 1)
        pen_ref[0] = jax.nn.sigmoid(logit - TAU * _C)

    @pl.when((j >= 1) & (j < 5))
    def _():
        s = jax.nn.sigmoid(logit / TAU)
        gate = jnp.clip(s * (ZETA - GAMMA) + GAMMA, 0.0, 1.0)
        weights_ref[...] = jnp.full((BWT,), gate, dtype=jnp.float32)

    v = s0_ref[...] + j * (BCE // K)

    @pl.when(j < GJ - 1)
    def _():
        edges_ref[...] = v

    @pl.when(j == GJ - 1)
    def _():
        edges_ref[...] = jnp.where(v >= N, v - N, v)


def kernel(x, batch, logit):
    del x, batch
    edges, weights, pen = pl.pallas_call(
        _gen_kernel,
        grid=(GJ,),
        in_specs=[pl.BlockSpec(memory_space=pltpu.SMEM)],
        out_specs=[
            pl.BlockSpec((2, BCE), lambda j: (0, j)),
            pl.BlockSpec((BWT,), lambda j: (jnp.clip(j - 1, 0, 3),)),
            pl.BlockSpec(memory_space=pltpu.SMEM),
        ],
        out_shape=[
            jax.ShapeDtypeStruct((2, E), jnp.int32),
            jax.ShapeDtypeStruct((E,), jnp.float32),
            jax.ShapeDtypeStruct((1,), jnp.float32),
        ],
        scratch_shapes=[pltpu.VMEM((2, BCE), jnp.int32)],
    )(logit)
    return edges, weights, pen.reshape(())
